# baseline (speedup 1.0000x reference)
import functools

import jax
import jax.numpy as jnp
from jax.experimental import pallas as pl
from jax.experimental.pallas import tpu as pltpu

T = 64
D = 1024
E = 64
DM = 512
DS = 512

NBUF = 2


def _mega_kernel(x_ref, gw_ref, segw_ref, gup_hbm, dp_hbm,
                 sg_hbm, su_hbm, sd_hbm, out_ref,
                 gbuf, dbuf, sgbuf, subuf, sdbuf,
                 ids_vmem, ids_smem, esem, ssem, isem):
    pltpu.make_async_copy(sg_hbm, sgbuf, ssem.at[0]).start()
    pltpu.make_async_copy(su_hbm, subuf, ssem.at[1]).start()
    pltpu.make_async_copy(sd_hbm, sdbuf, ssem.at[2]).start()

    x = x_ref[...]

    gw = gw_ref[...]
    logits = jax.lax.dot_general(
        x, gw, (((1,), (1,)), ((), ())), preferred_element_type=jnp.float32)
    m = jnp.max(logits, axis=1, keepdims=True)
    p = jnp.exp(logits - m)
    p = p / jnp.sum(p, axis=1, keepdims=True)
    lane = jax.lax.broadcasted_iota(jnp.int32, (T, E), 1)
    m1 = jnp.max(p, axis=1, keepdims=True)
    i1 = jnp.min(jnp.where(p >= m1, lane, E), axis=1, keepdims=True)
    oh1 = lane == i1
    p2 = jnp.where(oh1, -1.0, p)
    m2 = jnp.max(p2, axis=1, keepdims=True)
    i2 = jnp.min(jnp.where(p2 >= m2, lane, E), axis=1, keepdims=True)
    oh2 = lane == i2
    sden = m1 + m2 + 1e-20
    wf = jnp.where(oh1, m1 / sden, 0.0) + jnp.where(oh2, m2 / sden, 0.0)
    ohf = (oh1 | oh2).astype(jnp.float32)
    ones_t = jnp.ones((T, 1), jnp.float32)
    hits = jax.lax.dot_general(
        ohf, ones_t, (((0,), (0,)), ((), ())), preferred_element_type=jnp.float32)
    hit = (hits > 0).astype(jnp.float32)
    er = jax.lax.broadcasted_iota(jnp.int32, (E, E), 0)
    ec = jax.lax.broadcasted_iota(jnp.int32, (E, E), 1)
    ltri = (ec <= er).astype(jnp.float32)
    pos = jax.lax.dot_general(
        ltri, hit, (((1,), (0,)), ((), ())), preferred_element_type=jnp.float32)
    nv = jax.lax.dot_general(
        hit, ones_t[:E], (((0,), (0,)), ((), ())), preferred_element_type=jnp.float32)
    jlane = jax.lax.broadcasted_iota(jnp.int32, (E, E), 1).astype(jnp.float32)
    sel = ((pos - 1.0) == jlane).astype(jnp.float32) * hit
    evals = jax.lax.broadcasted_iota(jnp.int32, (E, 1), 0).astype(jnp.float32)
    eh = evals * hit
    ids = jax.lax.dot_general(
        sel, eh, (((0,), (0,)), ((), ())), preferred_element_type=jnp.float32)
    last = jnp.max(eh, axis=0, keepdims=True)
    jrow = jax.lax.broadcasted_iota(jnp.int32, (E, 1), 0).astype(jnp.float32)
    ids = jnp.where(jrow < nv, ids, last)
    ids_vmem[0:E, :] = ids.astype(jnp.int32)
    ids_vmem[E:E + 1, :] = nv.astype(jnp.int32)

    pltpu.make_async_copy(ids_vmem, ids_smem, isem).start()
    pltpu.make_async_copy(ids_vmem, ids_smem, isem).wait()

    n = ids_smem[E, 0]

    def issue(j):
        e = ids_smem[j, 0]
        s = jax.lax.rem(j, NBUF)
        pltpu.make_async_copy(gup_hbm.at[e], gbuf.at[s], esem.at[s, 0]).start()
        pltpu.make_async_copy(dp_hbm.at[e], dbuf.at[s], esem.at[s, 2]).start()

    def wait(j):
        e = ids_smem[j, 0]
        s = jax.lax.rem(j, NBUF)
        pltpu.make_async_copy(gup_hbm.at[e], gbuf.at[s], esem.at[s, 0]).wait()
        pltpu.make_async_copy(dp_hbm.at[e], dbuf.at[s], esem.at[s, 2]).wait()

    for k in range(NBUF - 1):
        @pl.when(k < n)
        def _():
            issue(k)

    pltpu.make_async_copy(sg_hbm, sgbuf, ssem.at[0]).wait()
    pltpu.make_async_copy(su_hbm, subuf, ssem.at[1]).wait()
    pltpu.make_async_copy(sd_hbm, sdbuf, ssem.at[2]).wait()
    g = jax.lax.dot_general(
        x, sgbuf[...], (((1,), (1,)), ((), ())),
        preferred_element_type=jnp.float32)
    u = jax.lax.dot_general(
        x, subuf[...], (((1,), (1,)), ((), ())),
        preferred_element_type=jnp.float32)
    h = jax.nn.silu(g) * u
    sh = jax.lax.dot_general(
        h, sdbuf[...], (((1,), (1,)), ((), ())),
        preferred_element_type=jnp.float32)
    gl = jax.lax.dot_general(
        x, segw_ref[...], (((1,), (1,)), ((), ())),
        preferred_element_type=jnp.float32)
    out_ref[...] = jax.nn.sigmoid(gl) * sh

    def body(j, _):
        @pl.when(j + NBUF - 1 < n)
        def _():
            issue(j + NBUF - 1)
        wait(j)
        s = jax.lax.rem(j, NBUF)
        g = jax.lax.dot_general(
            x, gbuf[s, pl.ds(0, DM)], (((1,), (1,)), ((), ())),
            preferred_element_type=jnp.float32)
        u = jax.lax.dot_general(
            x, gbuf[s, pl.ds(DM, DM)], (((1,), (1,)), ((), ())),
            preferred_element_type=jnp.float32)
        h = jax.nn.silu(g) * u
        y = jax.lax.dot_general(
            h, dbuf[s], (((1,), (1,)), ((), ())),
            preferred_element_type=jnp.float32)
        e = ids_smem[j, 0]
        wcol = jnp.sum(jnp.where(lane == e, wf, 0.0),
                       axis=1, keepdims=True)
        out_ref[...] += y * wcol
        return 0

    jax.lax.fori_loop(0, n, body, 0)


@functools.partial(jax.jit, static_argnames=())
def kernel(hidden_states, gate_w, gate_up_proj, down_proj,
           shared_gate_proj, shared_up_proj, shared_down_proj,
           shared_expert_gate_w):
    b, s, d = hidden_states.shape
    x = hidden_states.reshape(T, D)

    out = pl.pallas_call(
        _mega_kernel,
        grid=(1,),
        in_specs=[
            pl.BlockSpec((T, D), lambda i: (0, 0)),
            pl.BlockSpec((E, D), lambda i: (0, 0)),
            pl.BlockSpec((1, D), lambda i: (0, 0)),
            pl.BlockSpec(memory_space=pltpu.MemorySpace.HBM),
            pl.BlockSpec(memory_space=pltpu.MemorySpace.HBM),
            pl.BlockSpec(memory_space=pltpu.MemorySpace.HBM),
            pl.BlockSpec(memory_space=pltpu.MemorySpace.HBM),
            pl.BlockSpec(memory_space=pltpu.MemorySpace.HBM),
        ],
        out_specs=pl.BlockSpec((T, D), lambda i: (0, 0)),
        out_shape=jax.ShapeDtypeStruct((T, D), jnp.float32),
        scratch_shapes=[
            pltpu.VMEM((NBUF, 2 * DM, D), jnp.float32),
            pltpu.VMEM((NBUF, D, DM), jnp.float32),
            pltpu.VMEM((DS, D), jnp.float32),
            pltpu.VMEM((DS, D), jnp.float32),
            pltpu.VMEM((D, DS), jnp.float32),
            pltpu.VMEM((E + 1, 1), jnp.int32),
            pltpu.SMEM((E + 1, 1), jnp.int32),
            pltpu.SemaphoreType.DMA((NBUF, 3)),
            pltpu.SemaphoreType.DMA((3,)),
            pltpu.SemaphoreType.DMA,
        ],
    )(x, gate_w, shared_expert_gate_w, gate_up_proj, down_proj,
      shared_gate_proj, shared_up_proj, shared_down_proj)

    return out.reshape(b, s, d)


# speedup vs baseline: 1.1022x; 1.1022x over previous
import functools

import jax
import jax.numpy as jnp
from jax.experimental import pallas as pl
from jax.experimental.pallas import tpu as pltpu

T = 64
D = 1024
E = 64
DM = 512
DS = 512

NBUF = 3


def _mega_kernel(x_ref, gw_ref, segw_ref, gup_hbm, dp_hbm,
                 sg_hbm, su_hbm, sd_hbm, out_ref,
                 gbuf, dbuf, sgbuf, subuf, sdbuf,
                 ids_vmem, ids_smem, esem, ssem, isem):
    pltpu.make_async_copy(sg_hbm, sgbuf, ssem.at[0]).start()
    pltpu.make_async_copy(su_hbm, subuf, ssem.at[1]).start()
    pltpu.make_async_copy(sd_hbm, sdbuf, ssem.at[2]).start()

    x = x_ref[...]

    gw = gw_ref[...]
    logits = jax.lax.dot_general(
        x, gw, (((1,), (1,)), ((), ())), preferred_element_type=jnp.float32)
    m = jnp.max(logits, axis=1, keepdims=True)
    p = jnp.exp(logits - m)
    p = p / jnp.sum(p, axis=1, keepdims=True)
    lane = jax.lax.broadcasted_iota(jnp.int32, (T, E), 1)
    m1 = jnp.max(p, axis=1, keepdims=True)
    i1 = jnp.min(jnp.where(p >= m1, lane, E), axis=1, keepdims=True)
    oh1 = lane == i1
    p2 = jnp.where(oh1, -1.0, p)
    m2 = jnp.max(p2, axis=1, keepdims=True)
    i2 = jnp.min(jnp.where(p2 >= m2, lane, E), axis=1, keepdims=True)
    oh2 = lane == i2
    ohf = (oh1 | oh2).astype(jnp.float32)
    ones_t = jnp.ones((T, 1), jnp.float32)
    hits = jax.lax.dot_general(
        ohf, ones_t, (((0,), (0,)), ((), ())), preferred_element_type=jnp.float32)
    hit = (hits > 0).astype(jnp.float32)
    er = jax.lax.broadcasted_iota(jnp.int32, (E, E), 0)
    ec = jax.lax.broadcasted_iota(jnp.int32, (E, E), 1)
    ltri = (ec <= er).astype(jnp.float32)
    pos = jax.lax.dot_general(
        ltri, hit, (((1,), (0,)), ((), ())), preferred_element_type=jnp.float32)
    nv = jax.lax.dot_general(
        hit, ones_t[:E], (((0,), (0,)), ((), ())), preferred_element_type=jnp.float32)
    jlane = jax.lax.broadcasted_iota(jnp.int32, (E, E), 1).astype(jnp.float32)
    sel = ((pos - 1.0) == jlane).astype(jnp.float32) * hit
    evals = jax.lax.broadcasted_iota(jnp.int32, (E, 1), 0).astype(jnp.float32)
    eh = evals * hit
    ids = jax.lax.dot_general(
        sel, eh, (((0,), (0,)), ((), ())), preferred_element_type=jnp.float32)
    last = jnp.max(eh, axis=0, keepdims=True)
    jrow = jax.lax.broadcasted_iota(jnp.int32, (E, 1), 0).astype(jnp.float32)
    ids = jnp.where(jrow < nv, ids, last)
    ids_vmem[0:E, :] = ids.astype(jnp.int32)
    ids_vmem[E:E + 1, :] = nv.astype(jnp.int32)

    pltpu.make_async_copy(ids_vmem, ids_smem, isem).start()
    sden = m1 + m2 + 1e-20
    wf = jnp.where(oh1, m1 / sden, 0.0) + jnp.where(oh2, m2 / sden, 0.0)
    pltpu.make_async_copy(ids_vmem, ids_smem, isem).wait()

    n = ids_smem[E, 0]

    def issue(j):
        e = ids_smem[j, 0]
        s = jax.lax.rem(j, NBUF)
        pltpu.make_async_copy(gup_hbm.at[e], gbuf.at[s], esem.at[s, 0]).start()
        pltpu.make_async_copy(dp_hbm.at[e], dbuf.at[s], esem.at[s, 2]).start()

    def wait(j):
        e = ids_smem[j, 0]
        s = jax.lax.rem(j, NBUF)
        pltpu.make_async_copy(gup_hbm.at[e], gbuf.at[s], esem.at[s, 0]).wait()
        pltpu.make_async_copy(dp_hbm.at[e], dbuf.at[s], esem.at[s, 2]).wait()

    for k in range(NBUF - 1):
        @pl.when(k < n)
        def _():
            issue(k)

    pltpu.make_async_copy(sg_hbm, sgbuf, ssem.at[0]).wait()
    pltpu.make_async_copy(su_hbm, subuf, ssem.at[1]).wait()
    pltpu.make_async_copy(sd_hbm, sdbuf, ssem.at[2]).wait()
    g = jax.lax.dot_general(
        x, sgbuf[...], (((1,), (1,)), ((), ())),
        preferred_element_type=jnp.float32)
    u = jax.lax.dot_general(
        x, subuf[...], (((1,), (1,)), ((), ())),
        preferred_element_type=jnp.float32)
    h = jax.nn.silu(g) * u
    sh = jax.lax.dot_general(
        h, sdbuf[...], (((1,), (1,)), ((), ())),
        preferred_element_type=jnp.float32)
    gl = jax.lax.dot_general(
        x, segw_ref[...], (((1,), (1,)), ((), ())),
        preferred_element_type=jnp.float32)
    out_ref[...] = jax.nn.sigmoid(gl) * sh

    def body(j, _):
        @pl.when(j + NBUF - 1 < n)
        def _():
            issue(j + NBUF - 1)
        wait(j)
        s = jax.lax.rem(j, NBUF)
        g = jax.lax.dot_general(
            x, gbuf[s, pl.ds(0, DM)], (((1,), (1,)), ((), ())),
            preferred_element_type=jnp.float32)
        u = jax.lax.dot_general(
            x, gbuf[s, pl.ds(DM, DM)], (((1,), (1,)), ((), ())),
            preferred_element_type=jnp.float32)
        h = jax.nn.silu(g) * u
        y = jax.lax.dot_general(
            h, dbuf[s], (((1,), (1,)), ((), ())),
            preferred_element_type=jnp.float32)
        e = ids_smem[j, 0]
        wcol = jnp.sum(jnp.where(lane == e, wf, 0.0),
                       axis=1, keepdims=True)
        out_ref[...] += y * wcol
        return 0

    jax.lax.fori_loop(0, n, body, 0)


@functools.partial(jax.jit, static_argnames=())
def kernel(hidden_states, gate_w, gate_up_proj, down_proj,
           shared_gate_proj, shared_up_proj, shared_down_proj,
           shared_expert_gate_w):
    b, s, d = hidden_states.shape
    x = hidden_states.reshape(T, D)

    out = pl.pallas_call(
        _mega_kernel,
        grid=(1,),
        in_specs=[
            pl.BlockSpec((T, D), lambda i: (0, 0)),
            pl.BlockSpec((E, D), lambda i: (0, 0)),
            pl.BlockSpec((1, D), lambda i: (0, 0)),
            pl.BlockSpec(memory_space=pltpu.MemorySpace.HBM),
            pl.BlockSpec(memory_space=pltpu.MemorySpace.HBM),
            pl.BlockSpec(memory_space=pltpu.MemorySpace.HBM),
            pl.BlockSpec(memory_space=pltpu.MemorySpace.HBM),
            pl.BlockSpec(memory_space=pltpu.MemorySpace.HBM),
        ],
        out_specs=pl.BlockSpec((T, D), lambda i: (0, 0)),
        out_shape=jax.ShapeDtypeStruct((T, D), jnp.float32),
        scratch_shapes=[
            pltpu.VMEM((NBUF, 2 * DM, D), jnp.float32),
            pltpu.VMEM((NBUF, D, DM), jnp.float32),
            pltpu.VMEM((DS, D), jnp.float32),
            pltpu.VMEM((DS, D), jnp.float32),
            pltpu.VMEM((D, DS), jnp.float32),
            pltpu.VMEM((E + 1, 1), jnp.int32),
            pltpu.SMEM((E + 1, 1), jnp.int32),
            pltpu.SemaphoreType.DMA((NBUF, 3)),
            pltpu.SemaphoreType.DMA((3,)),
            pltpu.SemaphoreType.DMA,
        ],
    )(x, gate_w, shared_expert_gate_w, gate_up_proj, down_proj,
      shared_gate_proj, shared_up_proj, shared_down_proj)

    return out.reshape(b, s, d)


# speedup vs baseline: 1.1028x; 1.0006x over previous
import functools

import jax
import jax.numpy as jnp
from jax.experimental import pallas as pl
from jax.experimental.pallas import tpu as pltpu

T = 64
D = 1024
E = 64
DM = 512
DS = 512

NBUF = 3


def _mega_kernel(x_ref, gw_ref, segw_ref, gup_hbm, dp_hbm,
                 sg_hbm, su_hbm, sd_hbm, out_ref,
                 gbuf, dbuf, sgbuf, subuf, sdbuf,
                 ids_vmem, ids_smem, esem, ssem, isem):
    pltpu.make_async_copy(sg_hbm, sgbuf, ssem.at[0]).start()
    pltpu.make_async_copy(su_hbm, subuf, ssem.at[1]).start()
    pltpu.make_async_copy(sd_hbm, sdbuf, ssem.at[2]).start()

    x = x_ref[...]

    gw = gw_ref[...]
    logits = jax.lax.dot_general(
        x, gw, (((1,), (1,)), ((), ())), preferred_element_type=jnp.float32)
    m = jnp.max(logits, axis=1, keepdims=True)
    p = jnp.exp(logits - m)
    p = p / jnp.sum(p, axis=1, keepdims=True)
    lane = jax.lax.broadcasted_iota(jnp.int32, (T, E), 1)
    m1 = jnp.max(p, axis=1, keepdims=True)
    i1 = jnp.min(jnp.where(p >= m1, lane, E), axis=1, keepdims=True)
    oh1 = lane == i1
    p2 = jnp.where(oh1, -1.0, p)
    m2 = jnp.max(p2, axis=1, keepdims=True)
    i2 = jnp.min(jnp.where(p2 >= m2, lane, E), axis=1, keepdims=True)
    oh2 = lane == i2
    ohf = (oh1 | oh2).astype(jnp.float32)
    ones_t = jnp.ones((T, 1), jnp.float32)
    hits = jax.lax.dot_general(
        ohf, ones_t, (((0,), (0,)), ((), ())), preferred_element_type=jnp.float32)
    hit = (hits > 0).astype(jnp.float32)
    er = jax.lax.broadcasted_iota(jnp.int32, (E, E), 0)
    ec = jax.lax.broadcasted_iota(jnp.int32, (E, E), 1)
    ltri = (ec <= er).astype(jnp.float32)
    pos = jax.lax.dot_general(
        ltri, hit, (((1,), (0,)), ((), ())), preferred_element_type=jnp.float32)
    nv = jax.lax.dot_general(
        hit, ones_t[:E], (((0,), (0,)), ((), ())), preferred_element_type=jnp.float32)
    jlane = jax.lax.broadcasted_iota(jnp.int32, (E, E), 1).astype(jnp.float32)
    sel = ((pos - 1.0) == jlane).astype(jnp.float32) * hit
    evals = jax.lax.broadcasted_iota(jnp.int32, (E, 1), 0).astype(jnp.float32)
    eh = evals * hit
    ids = jax.lax.dot_general(
        sel, eh, (((0,), (0,)), ((), ())), preferred_element_type=jnp.float32)
    last = jnp.max(eh, axis=0, keepdims=True)
    jrow = jax.lax.broadcasted_iota(jnp.int32, (E, 1), 0).astype(jnp.float32)
    ids = jnp.where(jrow < nv, ids, last)
    ids_vmem[0:E, :] = ids.astype(jnp.int32)
    ids_vmem[E:E + 1, :] = nv.astype(jnp.int32)

    pltpu.make_async_copy(ids_vmem, ids_smem, isem).start()
    sden = m1 + m2 + 1e-20
    wf = jnp.where(oh1, m1 / sden, 0.0) + jnp.where(oh2, m2 / sden, 0.0)
    pltpu.make_async_copy(ids_vmem, ids_smem, isem).wait()

    n = ids_smem[E, 0]

    def issue(j):
        e = ids_smem[j, 0]
        s = jax.lax.rem(j, NBUF)
        pltpu.make_async_copy(gup_hbm.at[e], gbuf.at[s], esem.at[s, 0]).start()
        pltpu.make_async_copy(dp_hbm.at[e], dbuf.at[s], esem.at[s, 1]).start()

    def wait(j):
        e = ids_smem[j, 0]
        s = jax.lax.rem(j, NBUF)
        pltpu.make_async_copy(gup_hbm.at[e], gbuf.at[s], esem.at[s, 0]).wait()
        pltpu.make_async_copy(dp_hbm.at[e], dbuf.at[s], esem.at[s, 1]).wait()

    for k in range(NBUF - 1):
        @pl.when(k < n)
        def _():
            issue(k)

    pltpu.make_async_copy(sg_hbm, sgbuf, ssem.at[0]).wait()
    pltpu.make_async_copy(su_hbm, subuf, ssem.at[1]).wait()
    pltpu.make_async_copy(sd_hbm, sdbuf, ssem.at[2]).wait()
    g = jax.lax.dot_general(
        x, sgbuf[...], (((1,), (1,)), ((), ())),
        preferred_element_type=jnp.float32)
    u = jax.lax.dot_general(
        x, subuf[...], (((1,), (1,)), ((), ())),
        preferred_element_type=jnp.float32)
    h = jax.nn.silu(g) * u
    sh = jax.lax.dot_general(
        h, sdbuf[...], (((1,), (1,)), ((), ())),
        preferred_element_type=jnp.float32)
    gl = jax.lax.dot_general(
        x, segw_ref[...], (((1,), (1,)), ((), ())),
        preferred_element_type=jnp.float32)
    out_ref[...] = jax.nn.sigmoid(gl) * sh

    def body(j, _):
        @pl.when(j + NBUF - 1 < n)
        def _():
            issue(j + NBUF - 1)
        wait(j)
        s = jax.lax.rem(j, NBUF)
        g = jax.lax.dot_general(
            x, gbuf[s, pl.ds(0, DM)], (((1,), (1,)), ((), ())),
            preferred_element_type=jnp.float32)
        u = jax.lax.dot_general(
            x, gbuf[s, pl.ds(DM, DM)], (((1,), (1,)), ((), ())),
            preferred_element_type=jnp.float32)
        h = jax.nn.silu(g) * u
        y = jax.lax.dot_general(
            h, dbuf[s], (((1,), (1,)), ((), ())),
            preferred_element_type=jnp.float32)
        e = ids_smem[j, 0]
        wcol = jnp.sum(jnp.where(lane == e, wf, 0.0),
                       axis=1, keepdims=True)
        out_ref[...] += y * wcol
        return 0

    jax.lax.fori_loop(0, n, body, 0)


@functools.partial(jax.jit, static_argnames=())
def kernel(hidden_states, gate_w, gate_up_proj, down_proj,
           shared_gate_proj, shared_up_proj, shared_down_proj,
           shared_expert_gate_w):
    b, s, d = hidden_states.shape
    x = hidden_states.reshape(T, D)

    out = pl.pallas_call(
        _mega_kernel,
        grid=(1,),
        in_specs=[
            pl.BlockSpec((T, D), lambda i: (0, 0)),
            pl.BlockSpec((E, D), lambda i: (0, 0)),
            pl.BlockSpec((1, D), lambda i: (0, 0)),
            pl.BlockSpec(memory_space=pltpu.MemorySpace.HBM),
            pl.BlockSpec(memory_space=pltpu.MemorySpace.HBM),
            pl.BlockSpec(memory_space=pltpu.MemorySpace.HBM),
            pl.BlockSpec(memory_space=pltpu.MemorySpace.HBM),
            pl.BlockSpec(memory_space=pltpu.MemorySpace.HBM),
        ],
        out_specs=pl.BlockSpec((T, D), lambda i: (0, 0)),
        out_shape=jax.ShapeDtypeStruct((T, D), jnp.float32),
        scratch_shapes=[
            pltpu.VMEM((NBUF, 2 * DM, D), jnp.float32),
            pltpu.VMEM((NBUF, D, DM), jnp.float32),
            pltpu.VMEM((DS, D), jnp.float32),
            pltpu.VMEM((DS, D), jnp.float32),
            pltpu.VMEM((D, DS), jnp.float32),
            pltpu.VMEM((E + 1, 1), jnp.int32),
            pltpu.SMEM((E + 1, 1), jnp.int32),
            pltpu.SemaphoreType.DMA((NBUF, 2)),
            pltpu.SemaphoreType.DMA((3,)),
            pltpu.SemaphoreType.DMA,
        ],
    )(x, gate_w, shared_expert_gate_w, gate_up_proj, down_proj,
      shared_gate_proj, shared_up_proj, shared_down_proj)

    return out.reshape(b, s, d)
